# baseline (speedup 1.0000x reference)
import jax
import jax.numpy as jnp
from jax.experimental import pallas as pl
from jax.experimental.pallas import tpu as pltpu


def _conv_kernel(x_ref, w_ref, b_ref, y_ref):
    xb = x_ref[0]
    xb = xb.reshape(3, 32, 16, 32, 16)
    xb = xb.transpose(0, 2, 4, 1, 3)
    patches_t = xb.reshape(768, 1024)
    y = jax.lax.dot_general(w_ref[...], patches_t, (((1,), (0,)), ((), ())),
                            preferred_element_type=jnp.float32)
    y_ref[...] = y + b_ref[...]


def _score_kernel(y_ref, g_ref, be_ref, w2_ref, b2_ref, s_ref):
    y = y_ref[...]
    mean = jnp.mean(y, axis=1, keepdims=True)
    var = jnp.mean((y - mean) ** 2, axis=1, keepdims=True)
    z = (y - mean) / jnp.sqrt(var + 1e-5) * g_ref[...] + be_ref[...]
    z = jnp.maximum(z, 0.0)
    s = jax.lax.dot_general(w2_ref[...], z, (((1,), (0,)), ((), ())),
                            preferred_element_type=jnp.float32)
    s_ref[...] = s + b2_ref[0, 0]


def _topk_kernel(s_ref, out_ref):
    iota_r = jax.lax.broadcasted_iota(jnp.int32, (1024, 1024), 0)
    iota_c = jax.lax.broadcasted_iota(jnp.int32, (1024, 1024), 1)
    ltmask = iota_r < iota_c

    def body(b, carry):
        srow = s_ref[pl.ds(b, 1), :]
        scol = srow.reshape(1024, 1)
        gt = scol > srow
        eq = scol == srow
        beats = jnp.logical_or(gt, jnp.logical_and(eq, ltmask))
        rank = jnp.sum(beats.astype(jnp.int32), axis=0)
        rank_col = rank.reshape(1024, 1)
        eqr = rank_col == iota_c
        outrow = jnp.sum(jnp.where(eqr, iota_r, 0), axis=0)
        out_ref[pl.ds(b, 1), :] = outrow.reshape(1, 1024)
        return carry

    jax.lax.fori_loop(0, 64, body, 0)


def _conv_stage(x, Wmat, b1, interpret=False):
    return pl.pallas_call(
        _conv_kernel,
        grid=(64,),
        in_specs=[
            pl.BlockSpec((1, 3, 512, 512), lambda n: (n, 0, 0, 0)),
            pl.BlockSpec((32, 768), lambda n: (0, 0)),
            pl.BlockSpec((32, 1), lambda n: (0, 0)),
        ],
        out_specs=pl.BlockSpec((32, 1024), lambda n: (0, n)),
        out_shape=jax.ShapeDtypeStruct((32, 65536), jnp.float32),
        interpret=interpret,
    )(x, Wmat, b1)


def _score_stage(y, gamma, beta, w2row, b2, interpret=False):
    return pl.pallas_call(
        _score_kernel,
        grid=(1,),
        in_specs=[
            pl.BlockSpec((32, 65536), lambda i: (0, 0)),
            pl.BlockSpec((32, 1), lambda i: (0, 0)),
            pl.BlockSpec((32, 1), lambda i: (0, 0)),
            pl.BlockSpec((1, 32), lambda i: (0, 0)),
            pl.BlockSpec((1, 1), lambda i: (0, 0)),
        ],
        out_specs=pl.BlockSpec((1, 65536), lambda i: (0, 0)),
        out_shape=jax.ShapeDtypeStruct((1, 65536), jnp.float32),
        interpret=interpret,
    )(y, gamma, beta, w2row, b2)


def _topk_stage(s, interpret=False):
    return pl.pallas_call(
        _topk_kernel,
        grid=(1,),
        in_specs=[pl.BlockSpec((64, 1024), lambda i: (0, 0))],
        out_specs=pl.BlockSpec((64, 1024), lambda i: (0, 0)),
        out_shape=jax.ShapeDtypeStruct((64, 1024), jnp.int32),
        interpret=interpret,
    )(s)


def kernel(x, W1, b1, gamma, beta, W2, b2):
    Wmat = W1.reshape(32, 768)
    y = _conv_stage(x, Wmat, b1.reshape(32, 1))
    s = _score_stage(y, gamma.reshape(32, 1), beta.reshape(32, 1),
                     W2.reshape(1, 32), b2.reshape(1, 1))
    idx = _topk_stage(s.reshape(64, 1024))
    return idx[:, :819]
